# baseline (speedup 1.0000x reference)
import functools

import jax
import jax.numpy as jnp
import numpy as np
from jax.experimental import pallas as pl

_HIGH = jax.lax.Precision.DEFAULT


def _dot(a, b, prec=_HIGH):
    return jax.lax.dot_general(a, b, (((1,), (0,)), ((), ())),
                               precision=prec,
                               preferred_element_type=jnp.float32)


def _dot_t(a, b, prec=_HIGH):
    return jax.lax.dot_general(a, b, (((1,), (1,)), ((), ())),
                               precision=prec,
                               preferred_element_type=jnp.float32)


def _pool_body(x_ref, at_ref, o_ref):
    s = jnp.sum(x_ref[...], axis=1)
    o_ref[...] = _dot(s, at_ref[...])


def _block_body(p_ref, we_ref, wq_ref, wk_ref, wv_ref, wo_ref, g1_ref, b1n_ref,
                w1_ref, bb1_ref, w2_ref, bb2_ref, g2_ref, b2n_ref,
                scores_ref, gat_ref):
    p = p_ref[...]
    z = jax.nn.relu(_dot(p, we_ref[...]))

    def dists(feat):
        fs = feat[:64]
        fq = feat[64:]
        qn = jnp.sum(fq * fq, axis=1, keepdims=True)
        sn = jnp.sum(fs * fs, axis=1, keepdims=True)
        cross = _dot_t(fq, fs)
        return -(qn - 2.0 * cross + sn.reshape(1, 64))

    scores_ref[...] = dists(z)

    q = _dot(z, wq_ref[...])
    k = _dot(z, wk_ref[...])
    v = _dot(z, wv_ref[...])
    aggs = []
    for h in range(2):
        sl = slice(h * 256, (h + 1) * 256)
        logits = _dot_t(q[:, sl], k[:, sl]) * (1.0 / 16.0)
        m = jnp.max(logits, axis=1, keepdims=True)
        e = jnp.exp(logits - m)
        denom = jnp.sum(e, axis=1, keepdims=True) + 1e-16
        aggs.append(_dot(e / denom, v[:, sl]))
    agg = jnp.concatenate(aggs, axis=1)

    def layernorm(t, g, b):
        mu = jnp.mean(t, axis=1, keepdims=True)
        var = jnp.mean((t - mu) ** 2, axis=1, keepdims=True)
        return (t - mu) * jax.lax.rsqrt(var + 1e-5) * g + b

    h1 = layernorm(z + _dot(agg, wo_ref[...]), g1_ref[...], b1n_ref[...])
    mlp = _dot(jax.nn.relu(_dot(h1, w1_ref[...]) + bb1_ref[...]), w2_ref[...])
    h2 = layernorm(h1 + mlp + bb2_ref[...], g2_ref[...], b2n_ref[...])
    gat_ref[...] = dists(h2)


@functools.partial(jax.jit, static_argnames=())
def kernel(x, W_enc, Wq, Wk, Wv, Wo, ln1_g, ln1_b, W1, b1, W2, b2, ln2_g, ln2_b):
    ways, n_views = x.shape[0], x.shape[1]
    N = ways * n_views
    rows = N * 3 * 14

    x3 = x.reshape(rows, 16, 224)
    at = np.zeros((224, 14), dtype=np.float32)
    for j in range(14):
        at[16 * j:16 * (j + 1), j] = 1.0 / 256.0
    at = jnp.asarray(at)

    BB = 448
    grid = (rows // BB,)
    pooled = pl.pallas_call(
        _pool_body,
        grid=grid,
        in_specs=[
            pl.BlockSpec((BB, 16, 224), lambda i: (i, 0, 0)),
            pl.BlockSpec((224, 14), lambda i: (0, 0)),
        ],
        out_specs=pl.BlockSpec((BB, 14), lambda i: (i, 0)),
        out_shape=jax.ShapeDtypeStruct((rows, 14), jnp.float32),
    )(x3, at)
    p = pooled.reshape(N, 588)
    perm = np.concatenate([
        np.arange(ways) * n_views,
        (np.arange(ways)[:, None] * n_views + np.arange(1, n_views)[None, :]
         ).reshape(-1),
    ])
    p = p[jnp.asarray(perm)]

    row = lambda t: t.reshape(1, -1)
    scores, gat_scores = pl.pallas_call(
        _block_body,
        out_shape=(
            jax.ShapeDtypeStruct((N - ways, ways), jnp.float32),
            jax.ShapeDtypeStruct((N - ways, ways), jnp.float32),
        ),
    )(p, W_enc, Wq, Wk, Wv, Wo, row(ln1_g), row(ln1_b), W1, row(b1),
      W2, row(b2), row(ln2_g), row(ln2_b))

    y_query = jnp.repeat(jnp.arange(ways, dtype=jnp.int32), n_views - 1)
    return (scores, gat_scores, y_query)


# speedup vs baseline: 1.0044x; 1.0044x over previous
import functools

import jax
import jax.numpy as jnp
import numpy as np
from jax.experimental import pallas as pl

_HIGH = jax.lax.Precision.DEFAULT


def _dot(a, b, prec=_HIGH):
    return jax.lax.dot_general(a, b, (((1,), (0,)), ((), ())),
                               precision=prec,
                               preferred_element_type=jnp.float32)


def _dot_t(a, b, prec=_HIGH):
    return jax.lax.dot_general(a, b, (((1,), (1,)), ((), ())),
                               precision=prec,
                               preferred_element_type=jnp.float32)


def _pool_body(x_ref, at_ref, o_ref):
    s = jnp.sum(x_ref[...], axis=1)
    o_ref[...] = _dot(s, at_ref[...])


def _block_body(p_ref, we_ref, wq_ref, wk_ref, wv_ref, wo_ref, g1_ref, b1n_ref,
                w1_ref, bb1_ref, w2_ref, bb2_ref, g2_ref, b2n_ref,
                scores_ref, gat_ref):
    p = p_ref[...]
    z = jax.nn.relu(_dot(p, we_ref[...]))

    def dists(feat):
        fs = feat[:64]
        fq = feat[64:]
        qn = jnp.sum(fq * fq, axis=1, keepdims=True)
        sn = jnp.sum(fs * fs, axis=1, keepdims=True)
        cross = _dot_t(fq, fs)
        return -(qn - 2.0 * cross + sn.reshape(1, 64))

    scores_ref[...] = dists(z)

    q = _dot(z, wq_ref[...])
    k = _dot(z, wk_ref[...])
    v = _dot(z, wv_ref[...])
    aggs = []
    for h in range(2):
        sl = slice(h * 256, (h + 1) * 256)
        logits = _dot_t(q[:, sl], k[:, sl]) * (1.0 / 16.0)
        m = jnp.max(logits, axis=1, keepdims=True)
        e = jnp.exp(logits - m)
        denom = jnp.sum(e, axis=1, keepdims=True) + 1e-16
        aggs.append(_dot(e / denom, v[:, sl]))
    agg = jnp.concatenate(aggs, axis=1)

    def layernorm(t, g, b):
        mu = jnp.mean(t, axis=1, keepdims=True)
        var = jnp.mean((t - mu) ** 2, axis=1, keepdims=True)
        return (t - mu) * jax.lax.rsqrt(var + 1e-5) * g + b

    h1 = layernorm(z + _dot(agg, wo_ref[...]), g1_ref[...], b1n_ref[...])
    mlp = _dot(jax.nn.relu(_dot(h1, w1_ref[...]) + bb1_ref[...]), w2_ref[...])
    h2 = layernorm(h1 + mlp + bb2_ref[...], g2_ref[...], b2n_ref[...])
    gat_ref[...] = dists(h2)


@functools.partial(jax.jit, static_argnames=())
def kernel(x, W_enc, Wq, Wk, Wv, Wo, ln1_g, ln1_b, W1, b1, W2, b2, ln2_g, ln2_b):
    ways, n_views = x.shape[0], x.shape[1]
    N = ways * n_views
    rows = N * 3 * 14

    x3 = x.reshape(rows, 16, 224)
    at = np.zeros((224, 14), dtype=np.float32)
    for j in range(14):
        at[16 * j:16 * (j + 1), j] = 1.0 / 256.0
    at = jnp.asarray(at)

    BB = 960
    grid = (rows // BB,)
    pooled = pl.pallas_call(
        _pool_body,
        grid=grid,
        in_specs=[
            pl.BlockSpec((BB, 16, 224), lambda i: (i, 0, 0)),
            pl.BlockSpec((224, 14), lambda i: (0, 0)),
        ],
        out_specs=pl.BlockSpec((BB, 14), lambda i: (i, 0)),
        out_shape=jax.ShapeDtypeStruct((rows, 14), jnp.float32),
    )(x3, at)
    p = pooled.reshape(N, 588)
    perm = np.concatenate([
        np.arange(ways) * n_views,
        (np.arange(ways)[:, None] * n_views + np.arange(1, n_views)[None, :]
         ).reshape(-1),
    ])
    p = p[jnp.asarray(perm)]

    row = lambda t: t.reshape(1, -1)
    scores, gat_scores = pl.pallas_call(
        _block_body,
        out_shape=(
            jax.ShapeDtypeStruct((N - ways, ways), jnp.float32),
            jax.ShapeDtypeStruct((N - ways, ways), jnp.float32),
        ),
    )(p, W_enc, Wq, Wk, Wv, Wo, row(ln1_g), row(ln1_b), W1, row(b1),
      W2, row(b2), row(ln2_g), row(ln2_b))

    y_query = jnp.repeat(jnp.arange(ways, dtype=jnp.int32), n_views - 1)
    return (scores, gat_scores, y_query)


# speedup vs baseline: 1.0159x; 1.0115x over previous
import functools

import jax
import jax.numpy as jnp
import numpy as np
from jax.experimental import pallas as pl

_HIGH = jax.lax.Precision.DEFAULT


def _dot(a, b, prec=_HIGH):
    return jax.lax.dot_general(a, b, (((1,), (0,)), ((), ())),
                               precision=prec,
                               preferred_element_type=jnp.float32)


def _dot_t(a, b, prec=_HIGH):
    return jax.lax.dot_general(a, b, (((1,), (1,)), ((), ())),
                               precision=prec,
                               preferred_element_type=jnp.float32)


def _pool_body(xa_ref, xb_ref, at_ref, o_ref):
    at = at_ref[...]
    o_ref[:336] = _dot(jnp.sum(xa_ref[...], axis=1), at)
    o_ref[336:] = _dot(jnp.sum(xb_ref[...], axis=1), at)


def _block_body(p_ref, we_ref, wq_ref, wk_ref, wv_ref, wo_ref, g1_ref, b1n_ref,
                w1_ref, bb1_ref, w2_ref, bb2_ref, g2_ref, b2n_ref,
                scores_ref, gat_ref):
    p = p_ref[...]
    z = jax.nn.relu(_dot(p, we_ref[...]))

    def dists(feat):
        fs = feat[:64]
        fq = feat[64:]
        qn = jnp.sum(fq * fq, axis=1, keepdims=True)
        sn = jnp.sum(fs * fs, axis=1, keepdims=True)
        cross = _dot_t(fq, fs)
        return -(qn - 2.0 * cross + sn.reshape(1, 64))

    scores_ref[...] = dists(z)

    q = _dot(z, wq_ref[...])
    k = _dot(z, wk_ref[...])
    v = _dot(z, wv_ref[...])
    aggs = []
    for h in range(2):
        sl = slice(h * 256, (h + 1) * 256)
        logits = _dot_t(q[:, sl], k[:, sl]) * (1.0 / 16.0)
        m = jnp.max(logits, axis=1, keepdims=True)
        e = jnp.exp(logits - m)
        denom = jnp.sum(e, axis=1, keepdims=True) + 1e-16
        aggs.append(_dot(e / denom, v[:, sl]))
    agg = jnp.concatenate(aggs, axis=1)

    def layernorm(t, g, b):
        mu = jnp.mean(t, axis=1, keepdims=True)
        var = jnp.mean((t - mu) ** 2, axis=1, keepdims=True)
        return (t - mu) * jax.lax.rsqrt(var + 1e-5) * g + b

    h1 = layernorm(z + _dot(agg, wo_ref[...]), g1_ref[...], b1n_ref[...])
    mlp = _dot(jax.nn.relu(_dot(h1, w1_ref[...]) + bb1_ref[...]), w2_ref[...])
    h2 = layernorm(h1 + mlp + bb2_ref[...], g2_ref[...], b2n_ref[...])
    gat_ref[...] = dists(h2)


@functools.partial(jax.jit, static_argnames=())
def kernel(x, W_enc, Wq, Wk, Wv, Wo, ln1_g, ln1_b, W1, b1, W2, b2, ln2_g, ln2_b):
    ways, n_views = x.shape[0], x.shape[1]
    N = ways * n_views
    rows = N * 3 * 14

    x3 = x.reshape(rows, 16, 224)
    at = np.zeros((224, 14), dtype=np.float32)
    for j in range(14):
        at[16 * j:16 * (j + 1), j] = 1.0 / 256.0
    at = jnp.asarray(at)

    BB = 672
    grid = (rows // BB,)
    pooled = pl.pallas_call(
        _pool_body,
        grid=grid,
        in_specs=[
            pl.BlockSpec((BB // 2, 16, 224), lambda i: (2 * i, 0, 0)),
            pl.BlockSpec((BB // 2, 16, 224), lambda i: (2 * i + 1, 0, 0)),
            pl.BlockSpec((224, 14), lambda i: (0, 0)),
        ],
        out_specs=pl.BlockSpec((BB, 14), lambda i: (i, 0)),
        out_shape=jax.ShapeDtypeStruct((rows, 14), jnp.float32),
    )(x3, x3, at)
    p = pooled.reshape(N, 588)
    perm = np.concatenate([
        np.arange(ways) * n_views,
        (np.arange(ways)[:, None] * n_views + np.arange(1, n_views)[None, :]
         ).reshape(-1),
    ])
    p = p[jnp.asarray(perm)]

    row = lambda t: t.reshape(1, -1)
    scores, gat_scores = pl.pallas_call(
        _block_body,
        out_shape=(
            jax.ShapeDtypeStruct((N - ways, ways), jnp.float32),
            jax.ShapeDtypeStruct((N - ways, ways), jnp.float32),
        ),
    )(p, W_enc, Wq, Wk, Wv, Wo, row(ln1_g), row(ln1_b), W1, row(b1),
      W2, row(b2), row(ln2_g), row(ln2_b))

    y_query = jnp.repeat(jnp.arange(ways, dtype=jnp.int32), n_views - 1)
    return (scores, gat_scores, y_query)


# speedup vs baseline: 1.1329x; 1.1151x over previous
import functools

import jax
import jax.numpy as jnp
import numpy as np
from jax.experimental import pallas as pl
from jax.experimental.pallas import tpu as pltpu

_BB = 672
_ROWS = 13440
_STEPS = _ROWS // _BB
_SPB = _BB // 42


def _dot(a, b):
    return jax.lax.dot_general(a, b, (((1,), (0,)), ((), ())),
                               preferred_element_type=jnp.float32)


def _dot_t(a, b):
    return jax.lax.dot_general(a, b, (((1,), (1,)), ((), ())),
                               preferred_element_type=jnp.float32)


def _body(x_ref, at_ref, h_ref, mask_ref, s16_ref, we_ref, wq_ref, wk_ref,
          wv_ref, wo_ref, g1_ref, b1n_ref, w1_ref, bb1_ref, w2_ref, bb2_ref,
          g2_ref, b2n_ref, sels_ref, selq_ref, scores_ref, gat_ref, p_scr):
    i = pl.program_id(0)

    @pl.when(i < _STEPS)
    def _pool():
        s = jnp.sum(x_ref[...], axis=1)
        pr = _dot(s, at_ref[...])
        tiled = _dot(pr, h_ref[...]) * mask_ref[...]
        p_scr[pl.ds(i * _SPB, _SPB), :] = _dot(s16_ref[...], tiled)

    @pl.when(i == _STEPS)
    def _block():
        z = jax.nn.relu(_dot(p_scr[...], we_ref[...]))

        def dists(feat):
            fs = _dot(sels_ref[...], feat)
            fq = _dot(selq_ref[...], feat)
            qn = jnp.sum(fq * fq, axis=1, keepdims=True)
            sn = jnp.sum(fs * fs, axis=1, keepdims=True)
            return -(qn - 2.0 * _dot_t(fq, fs) + sn.reshape(1, 64))

        scores_ref[...] = dists(z)

        q = _dot(z, wq_ref[...])
        k = _dot(z, wk_ref[...])
        v = _dot(z, wv_ref[...])
        aggs = []
        for h in range(2):
            sl = slice(h * 256, (h + 1) * 256)
            logits = _dot_t(q[:, sl], k[:, sl]) * (1.0 / 16.0)
            m = jnp.max(logits, axis=1, keepdims=True)
            e = jnp.exp(logits - m)
            denom = jnp.sum(e, axis=1, keepdims=True) + 1e-16
            aggs.append(_dot(e / denom, v[:, sl]))
        agg = jnp.concatenate(aggs, axis=1)

        def layernorm(t, g, b):
            mu = jnp.mean(t, axis=1, keepdims=True)
            var = jnp.mean((t - mu) ** 2, axis=1, keepdims=True)
            return (t - mu) * jax.lax.rsqrt(var + 1e-5) * g + b

        h1 = layernorm(z + _dot(agg, wo_ref[...]), g1_ref[...], b1n_ref[...])
        mlp = _dot(jax.nn.relu(_dot(h1, w1_ref[...]) + bb1_ref[...]),
                   w2_ref[...])
        h2 = layernorm(h1 + mlp + bb2_ref[...], g2_ref[...], b2n_ref[...])
        gat_ref[...] = dists(h2)


@functools.partial(jax.jit, static_argnames=())
def kernel(x, W_enc, Wq, Wk, Wv, Wo, ln1_g, ln1_b, W1, b1, W2, b2, ln2_g, ln2_b):
    ways, n_views = x.shape[0], x.shape[1]
    N = ways * n_views

    x3 = x.reshape(_ROWS, 16, 224)
    at = np.zeros((224, 14), dtype=np.float32)
    for j in range(14):
        at[16 * j:16 * (j + 1), j] = 1.0 / 256.0

    h = np.tile(np.eye(14, dtype=np.float32), (1, 42))
    rr = np.arange(_BB) % 42
    mask = (np.arange(588)[None, :] // 14 == rr[:, None]).astype(np.float32)
    s16 = np.zeros((_SPB, _BB), dtype=np.float32)
    for n in range(_SPB):
        s16[n, n * 42:(n + 1) * 42] = 1.0

    nat = np.arange(N)
    sup = nat[nat % n_views == 0]
    qry = nat[nat % n_views != 0]
    sels = np.zeros((ways, N), dtype=np.float32)
    sels[np.arange(ways), sup] = 1.0
    selq = np.zeros((N - ways, N), dtype=np.float32)
    selq[np.arange(N - ways), qry] = 1.0

    row = lambda t: t.reshape(1, -1)
    const = lambda *shape: pl.BlockSpec(shape, lambda i: (0,) * len(shape))
    scores, gat_scores = pl.pallas_call(
        _body,
        grid=(_STEPS + 1,),
        in_specs=[
            pl.BlockSpec((_BB, 16, 224),
                         lambda i: (jnp.minimum(i, _STEPS - 1), 0, 0)),
            const(224, 14), const(14, 588), const(_BB, 588),
            const(_SPB, _BB), const(588, 512),
            const(512, 512), const(512, 512), const(512, 512),
            const(512, 512), const(1, 512), const(1, 512),
            const(512, 512), const(1, 512), const(512, 512), const(1, 512),
            const(1, 512), const(1, 512),
            const(ways, N), const(N - ways, N),
        ],
        out_specs=[
            pl.BlockSpec((N - ways, ways), lambda i: (0, 0)),
            pl.BlockSpec((N - ways, ways), lambda i: (0, 0)),
        ],
        out_shape=(
            jax.ShapeDtypeStruct((N - ways, ways), jnp.float32),
            jax.ShapeDtypeStruct((N - ways, ways), jnp.float32),
        ),
        scratch_shapes=[pltpu.VMEM((N, 588), jnp.float32)],
    )(x3, jnp.asarray(at), jnp.asarray(h), jnp.asarray(mask),
      jnp.asarray(s16), W_enc, Wq, Wk, Wv, Wo, row(ln1_g), row(ln1_b),
      W1, row(b1), W2, row(b2), row(ln2_g), row(ln2_b),
      jnp.asarray(sels), jnp.asarray(selq))

    y_query = jnp.repeat(jnp.arange(ways, dtype=jnp.int32), n_views - 1)
    return (scores, gat_scores, y_query)
